# baseline (speedup 1.0000x reference)
import functools

import jax
import jax.numpy as jnp
from jax import lax
from jax.experimental import pallas as pl
from jax.experimental.pallas import tpu as pltpu
from jax.experimental.pallas import tpu_sc as plsc

MEM_DIM = 128
N_ID = 16384

_NC = 2
_NS = 16
_NW = _NC * _NS
_B_PER_W = N_ID // _NW
_CHUNK = 128
_N_CHUNKS = _B_PER_W // _CHUNK


def _sc_gather_body(n_id_hbm, mem_hbm, lu_hbm, h_out, lu_out,
                    idx_v, rows_v, lu_v, sem, sem_lu):
    wid = lax.axis_index("s") * _NC + lax.axis_index("c")
    base = wid * _B_PER_W
    pltpu.sync_copy(n_id_hbm.at[pl.ds(base, _B_PER_W)], idx_v)
    row_cps = []
    lu_cps = []
    for c in range(_N_CHUNKS):
        idx_c = idx_v.at[pl.ds(c * _CHUNK, _CHUNK)]
        row_cps.append(pltpu.async_copy(mem_hbm.at[idx_c], rows_v.at[c], sem))
        lu_cps.append(pltpu.async_copy(lu_hbm.at[idx_c], lu_v.at[c], sem_lu))
    for c in range(_N_CHUNKS):
        off = base + c * _CHUNK
        row_cps[c].wait()
        pltpu.sync_copy(rows_v.at[c], h_out.at[pl.ds(off, _CHUNK)])
        lu_cps[c].wait()
        pltpu.sync_copy(lu_v.at[c], lu_out.at[pl.ds(off, _CHUNK)])


_sc_gather = functools.partial(
    pl.kernel,
    mesh=plsc.VectorSubcoreMesh(core_axis_name="c", subcore_axis_name="s"),
    out_type=[
        jax.ShapeDtypeStruct((N_ID, MEM_DIM), jnp.float32),
        jax.ShapeDtypeStruct((N_ID,), jnp.int32),
    ],
    scratch_types=[
        pltpu.VMEM((_B_PER_W,), jnp.int32),
        pltpu.VMEM((_N_CHUNKS, _CHUNK, MEM_DIM), jnp.float32),
        pltpu.VMEM((_N_CHUNKS, _CHUNK), jnp.int32),
        pltpu.SemaphoreType.DMA,
        pltpu.SemaphoreType.DMA,
    ],
)(_sc_gather_body)


_BLK = 2048


def _gru_body(h_ref, wr_ref, wz_ref, wn_ref, br_ref, bz_ref, bni_ref, bnh_ref,
              out_ref):
    h = h_ref[...]
    dn = (((1,), (0,)), ((), ()))
    r = jax.nn.sigmoid(
        jax.lax.dot_general(h, wr_ref[...], dn,
                            preferred_element_type=jnp.float32) + br_ref[...])
    z = jax.nn.sigmoid(
        jax.lax.dot_general(h, wz_ref[...], dn,
                            preferred_element_type=jnp.float32) + bz_ref[...])
    ghn = jax.lax.dot_general(h, wn_ref[...], dn,
                              preferred_element_type=jnp.float32) + bnh_ref[...]
    n = jnp.tanh(bni_ref[...] + r * ghn)
    out_ref[...] = n + z * (h - n)


def _gru(h, wr, wz, wn, br, bz, bni, bnh):
    grid = N_ID // _BLK
    w_spec = pl.BlockSpec((MEM_DIM, MEM_DIM), lambda i: (0, 0))
    b_spec = pl.BlockSpec((1, MEM_DIM), lambda i: (0, 0))
    return pl.pallas_call(
        _gru_body,
        grid=(grid,),
        in_specs=[
            pl.BlockSpec((_BLK, MEM_DIM), lambda i: (i, 0)),
            w_spec, w_spec, w_spec,
            b_spec, b_spec, b_spec, b_spec,
        ],
        out_specs=pl.BlockSpec((_BLK, MEM_DIM), lambda i: (i, 0)),
        out_shape=jax.ShapeDtypeStruct((N_ID, MEM_DIM), jnp.float32),
    )(h, wr, wz, wn, br, bz, bni, bnh)


@jax.jit
def kernel(n_id, memory, last_update, W_ih, W_hh, b_ih, b_hh):
    del W_ih
    h, lu_out = _sc_gather(n_id.astype(jnp.int32), memory,
                           last_update.astype(jnp.int32))
    D = MEM_DIM
    wr = W_hh[:D].T
    wz = W_hh[D:2 * D].T
    wn = W_hh[2 * D:].T
    br = (b_ih[:D] + b_hh[:D]).reshape(1, D)
    bz = (b_ih[D:2 * D] + b_hh[D:2 * D]).reshape(1, D)
    bni = b_ih[2 * D:].reshape(1, D)
    bnh = b_hh[2 * D:].reshape(1, D)
    mem_out = _gru(h, wr, wz, wn, br, bz, bni, bnh)
    return (mem_out, lu_out.astype(last_update.dtype))


# speedup vs baseline: 1.0425x; 1.0425x over previous
import functools

import jax
import jax.numpy as jnp
from jax import lax
from jax.experimental import pallas as pl
from jax.experimental.pallas import tpu as pltpu
from jax.experimental.pallas import tpu_sc as plsc

MEM_DIM = 128
N_ID = 16384

_NC = 2
_NS = 16
_NW = _NC * _NS
_B_PER_W = N_ID // _NW
_CHUNK = 128
_N_CHUNKS = _B_PER_W // _CHUNK


def _sc_gather_body(n_id_hbm, mem_hbm, lu_hbm, h_out, lu_out,
                    idx_v, rows_v, lu_v, sem, sem_lu):
    wid = lax.axis_index("s") * _NC + lax.axis_index("c")
    base = wid * _B_PER_W
    pltpu.sync_copy(n_id_hbm.at[pl.ds(base, _B_PER_W)], idx_v)
    row_cps = []
    lu_cps = []
    for c in range(_N_CHUNKS):
        idx_c = idx_v.at[pl.ds(c * _CHUNK, _CHUNK)]
        row_cps.append(pltpu.async_copy(mem_hbm.at[idx_c], rows_v.at[c], sem))
        lu_cps.append(pltpu.async_copy(lu_hbm.at[idx_c], lu_v.at[c], sem_lu))
    for c in range(_N_CHUNKS):
        off = base + c * _CHUNK
        row_cps[c].wait()
        pltpu.sync_copy(rows_v.at[c], h_out.at[pl.ds(off, _CHUNK)])
        lu_cps[c].wait()
        pltpu.sync_copy(lu_v.at[c], lu_out.at[pl.ds(off, _CHUNK)])


_sc_gather = functools.partial(
    pl.kernel,
    mesh=plsc.VectorSubcoreMesh(core_axis_name="c", subcore_axis_name="s"),
    out_type=[
        jax.ShapeDtypeStruct((N_ID, MEM_DIM), jnp.float32),
        jax.ShapeDtypeStruct((N_ID,), jnp.int32),
    ],
    scratch_types=[
        pltpu.VMEM((_B_PER_W,), jnp.int32),
        pltpu.VMEM((_N_CHUNKS, _CHUNK, MEM_DIM), jnp.float32),
        pltpu.VMEM((_N_CHUNKS, _CHUNK), jnp.int32),
        pltpu.SemaphoreType.DMA,
        pltpu.SemaphoreType.DMA,
    ],
)(_sc_gather_body)


_BLK = 4096


def _gru_body(h_ref, wr_ref, wz_ref, wn_ref, br_ref, bz_ref, bni_ref, bnh_ref,
              out_ref):
    h = h_ref[...]
    dn = (((1,), (0,)), ((), ()))
    r = jax.nn.sigmoid(
        jax.lax.dot_general(h, wr_ref[...], dn,
                            preferred_element_type=jnp.float32) + br_ref[...])
    z = jax.nn.sigmoid(
        jax.lax.dot_general(h, wz_ref[...], dn,
                            preferred_element_type=jnp.float32) + bz_ref[...])
    ghn = jax.lax.dot_general(h, wn_ref[...], dn,
                              preferred_element_type=jnp.float32) + bnh_ref[...]
    n = jnp.tanh(bni_ref[...] + r * ghn)
    out_ref[...] = n + z * (h - n)


def _gru(h, wr, wz, wn, br, bz, bni, bnh):
    grid = N_ID // _BLK
    w_spec = pl.BlockSpec((MEM_DIM, MEM_DIM), lambda i: (0, 0))
    b_spec = pl.BlockSpec((1, MEM_DIM), lambda i: (0, 0))
    return pl.pallas_call(
        _gru_body,
        grid=(grid,),
        in_specs=[
            pl.BlockSpec((_BLK, MEM_DIM), lambda i: (i, 0)),
            w_spec, w_spec, w_spec,
            b_spec, b_spec, b_spec, b_spec,
        ],
        out_specs=pl.BlockSpec((_BLK, MEM_DIM), lambda i: (i, 0)),
        out_shape=jax.ShapeDtypeStruct((N_ID, MEM_DIM), jnp.float32),
    )(h, wr, wz, wn, br, bz, bni, bnh)


@jax.jit
def kernel(n_id, memory, last_update, W_ih, W_hh, b_ih, b_hh):
    del W_ih
    h, lu_out = _sc_gather(n_id.astype(jnp.int32), memory,
                           last_update.astype(jnp.int32))
    D = MEM_DIM
    wr = W_hh[:D].T
    wz = W_hh[D:2 * D].T
    wn = W_hh[2 * D:].T
    br = (b_ih[:D] + b_hh[:D]).reshape(1, D)
    bz = (b_ih[D:2 * D] + b_hh[D:2 * D]).reshape(1, D)
    bni = b_ih[2 * D:].reshape(1, D)
    bnh = b_hh[2 * D:].reshape(1, D)
    mem_out = _gru(h, wr, wz, wn, br, bz, bni, bnh)
    return (mem_out, lu_out.astype(last_update.dtype))


# speedup vs baseline: 1.0481x; 1.0053x over previous
import functools

import jax
import jax.numpy as jnp
from jax import lax
from jax.experimental import pallas as pl
from jax.experimental.pallas import tpu as pltpu
from jax.experimental.pallas import tpu_sc as plsc

MEM_DIM = 128
N_ID = 16384

_NC = 2
_NS = 16
_NW = _NC * _NS
_B_PER_W = N_ID // _NW
_CHUNK = 128
_N_CHUNKS = _B_PER_W // _CHUNK


def _sc_gather_body(n_id_hbm, mem_hbm, lu_hbm, h_out, lu_out,
                    idx_v, rows_v, lu_v, sem, sem_lu):
    wid = lax.axis_index("s") * _NC + lax.axis_index("c")
    base = wid * _B_PER_W
    pltpu.sync_copy(n_id_hbm.at[pl.ds(base, _B_PER_W)], idx_v)
    row_cps = []
    lu_cps = []
    for c in range(_N_CHUNKS):
        idx_c = idx_v.at[pl.ds(c * _CHUNK, _CHUNK)]
        row_cps.append(pltpu.async_copy(
            mem_hbm.at[idx_c], rows_v.at[pl.ds(c * _CHUNK, _CHUNK)], sem))
        lu_cps.append(pltpu.async_copy(
            lu_hbm.at[idx_c], lu_v.at[pl.ds(c * _CHUNK, _CHUNK)], sem_lu))
    for cp in row_cps:
        cp.wait()
    pltpu.sync_copy(rows_v, h_out.at[pl.ds(base, _B_PER_W)])
    for cp in lu_cps:
        cp.wait()
    pltpu.sync_copy(lu_v, lu_out.at[pl.ds(base, _B_PER_W)])


_sc_gather = functools.partial(
    pl.kernel,
    mesh=plsc.VectorSubcoreMesh(core_axis_name="c", subcore_axis_name="s"),
    out_type=[
        jax.ShapeDtypeStruct((N_ID, MEM_DIM), jnp.float32),
        jax.ShapeDtypeStruct((N_ID,), jnp.int32),
    ],
    scratch_types=[
        pltpu.VMEM((_B_PER_W,), jnp.int32),
        pltpu.VMEM((_B_PER_W, MEM_DIM), jnp.float32),
        pltpu.VMEM((_B_PER_W,), jnp.int32),
        pltpu.SemaphoreType.DMA,
        pltpu.SemaphoreType.DMA,
    ],
)(_sc_gather_body)


_BLK = 8192


def _gru_body(h_ref, wr_ref, wz_ref, wn_ref, br_ref, bz_ref, bni_ref, bnh_ref,
              out_ref):
    h = h_ref[...]
    dn = (((1,), (0,)), ((), ()))
    r = jax.nn.sigmoid(
        jax.lax.dot_general(h, wr_ref[...], dn,
                            preferred_element_type=jnp.float32) + br_ref[...])
    z = jax.nn.sigmoid(
        jax.lax.dot_general(h, wz_ref[...], dn,
                            preferred_element_type=jnp.float32) + bz_ref[...])
    ghn = jax.lax.dot_general(h, wn_ref[...], dn,
                              preferred_element_type=jnp.float32) + bnh_ref[...]
    n = jnp.tanh(bni_ref[...] + r * ghn)
    out_ref[...] = n + z * (h - n)


def _gru(h, wr, wz, wn, br, bz, bni, bnh):
    grid = N_ID // _BLK
    w_spec = pl.BlockSpec((MEM_DIM, MEM_DIM), lambda i: (0, 0))
    b_spec = pl.BlockSpec((1, MEM_DIM), lambda i: (0, 0))
    return pl.pallas_call(
        _gru_body,
        grid=(grid,),
        in_specs=[
            pl.BlockSpec((_BLK, MEM_DIM), lambda i: (i, 0)),
            w_spec, w_spec, w_spec,
            b_spec, b_spec, b_spec, b_spec,
        ],
        out_specs=pl.BlockSpec((_BLK, MEM_DIM), lambda i: (i, 0)),
        out_shape=jax.ShapeDtypeStruct((N_ID, MEM_DIM), jnp.float32),
    )(h, wr, wz, wn, br, bz, bni, bnh)


@jax.jit
def kernel(n_id, memory, last_update, W_ih, W_hh, b_ih, b_hh):
    del W_ih
    h, lu_out = _sc_gather(n_id.astype(jnp.int32), memory,
                           last_update.astype(jnp.int32))
    D = MEM_DIM
    wr = W_hh[:D].T
    wz = W_hh[D:2 * D].T
    wn = W_hh[2 * D:].T
    br = (b_ih[:D] + b_hh[:D]).reshape(1, D)
    bz = (b_ih[D:2 * D] + b_hh[D:2 * D]).reshape(1, D)
    bni = b_ih[2 * D:].reshape(1, D)
    bnh = b_hh[2 * D:].reshape(1, D)
    mem_out = _gru(h, wr, wz, wn, br, bz, bni, bnh)
    return (mem_out, lu_out.astype(last_update.dtype))
